# baseline (speedup 1.0000x reference)
import functools

import jax
import jax.numpy as jnp
from jax import lax
from jax.experimental import pallas as pl
from jax.experimental.pallas import tpu as pltpu
from jax.experimental.pallas import tpu_sc as plsc

VOCAB = 1000
EMBED = 300
BATCH = 1024
HIST = 200

SPLIT = 256
TAIL = EMBED - SPLIT
TAIL_PAD = 128

B_TOTAL = BATCH * HIST
NUM_CORES = 2
NUM_SUBCORES = 16
NW = NUM_CORES * NUM_SUBCORES
B_PER_W = B_TOTAL // NW
BATCH_PER_W = BATCH // NW
CHUNK = 40
CHUNKS_PER_H = HIST // CHUNK
N_CHUNKS = B_PER_W // CHUNK
NBUF = 2


def _gather_body(
    ta_hbm, tb_hbm, idx_hbm, out_hbm,
    idx_v, rows_n, rows_t,
    gasem0, gasem1, gbsem0, gbsem1, wsem0, wsem1,
):
    gasems = (gasem0, gasem1)
    gbsems = (gbsem0, gbsem1)
    wsems = (wsem0, wsem1)
    wid = lax.axis_index("s") * NUM_CORES + lax.axis_index("c")
    base = wid * B_PER_W
    bbase = wid * BATCH_PER_W

    pltpu.sync_copy(idx_hbm.at[pl.ds(base, B_PER_W)], idx_v)

    def start_gathers(g, b):
        idx_slice = idx_v.at[pl.ds(g * CHUNK, CHUNK)]
        pltpu.async_copy(
            ta_hbm.at[idx_slice], rows_n.at[b, :, pl.ds(0, SPLIT)], gasems[b]
        )
        pltpu.async_copy(tb_hbm.at[idx_slice], rows_t.at[b], gbsems[b])

    def wait_gathers(g, b):
        idx_slice = idx_v.at[pl.ds(g * CHUNK, CHUNK)]
        pltpu.make_async_copy(
            ta_hbm.at[idx_slice], rows_n.at[b, :, pl.ds(0, SPLIT)], gasems[b]
        ).wait()
        pltpu.make_async_copy(
            tb_hbm.at[idx_slice], rows_t.at[b], gbsems[b]
        ).wait()

    def out_slice(g):
        return out_hbm.at[
            bbase + g // CHUNKS_PER_H, pl.ds((g % CHUNKS_PER_H) * CHUNK, CHUNK)
        ]

    def wait_wb(b):
        pltpu.make_async_copy(rows_n.at[b], out_slice(0), wsems[b]).wait()

    start_gathers(0, 0)

    def outer(i, carry):
        for b in range(NBUF):
            g = i * NBUF + b
            nb = (b + 1) % NBUF

            @pl.when(g + 1 < N_CHUNKS)
            def _():
                @pl.when(g >= 1)
                def _():
                    wait_wb(nb)

                start_gathers(g + 1, nb)

            wait_gathers(g, b)

            def row_body(r, carry2):
                rows_n[b, r, pl.ds(SPLIT, 16)] = rows_t[b, r, pl.ds(0, 16)]
                rows_n[b, r, pl.ds(SPLIT + 16, 16)] = rows_t[b, r, pl.ds(16, 16)]
                lanes = lax.iota(jnp.int32, 16)
                vals = rows_t[b, r, pl.ds(32, 16)]
                plsc.store_scatter(
                    rows_n,
                    [jnp.full((16,), b, jnp.int32),
                     jnp.full((16,), r, jnp.int32),
                     SPLIT + 32 + lanes],
                    vals,
                    mask=lanes < TAIL - 32,
                )
                return carry2

            lax.fori_loop(0, CHUNK, row_body, 0)

            pltpu.async_copy(rows_n.at[b], out_slice(g), wsems[b])
        return carry

    lax.fori_loop(0, N_CHUNKS // NBUF, outer, 0)

    for b in range(NBUF):
        wait_wb(b)


@jax.jit
def _sc_gather(table_a, table_b, idx_flat):
    k = functools.partial(
        pl.kernel,
        out_type=jax.ShapeDtypeStruct((BATCH, HIST, EMBED), jnp.float32),
        mesh=plsc.VectorSubcoreMesh(core_axis_name="c", subcore_axis_name="s"),
        scratch_types=[
            pltpu.VMEM((B_PER_W,), jnp.int32),
            pltpu.VMEM((NBUF, CHUNK, EMBED), jnp.float32),
            pltpu.VMEM((NBUF, CHUNK, TAIL_PAD), jnp.float32),
            pltpu.SemaphoreType.DMA,
            pltpu.SemaphoreType.DMA,
            pltpu.SemaphoreType.DMA,
            pltpu.SemaphoreType.DMA,
            pltpu.SemaphoreType.DMA,
            pltpu.SemaphoreType.DMA,
        ],
        compiler_params=pltpu.CompilerParams(needs_layout_passes=False),
    )(_gather_body)
    return k(table_a, table_b, idx_flat)


def kernel(table, x):
    idx_flat = x.reshape(B_TOTAL)
    table_a = table[:, :SPLIT]
    table_b = jnp.pad(table[:, SPLIT:], ((0, 0), (0, TAIL_PAD - TAIL)))
    return _sc_gather(table_a, table_b, idx_flat)


# speedup vs baseline: 1.1367x; 1.1367x over previous
import functools

import jax
import jax.numpy as jnp
from jax import lax
from jax.experimental import pallas as pl
from jax.experimental.pallas import tpu as pltpu
from jax.experimental.pallas import tpu_sc as plsc

VOCAB = 1000
EMBED = 300
BATCH = 1024
HIST = 200

SPLIT = 256
TAIL = EMBED - SPLIT
TAIL_PAD = 128

B_TOTAL = BATCH * HIST
NUM_CORES = 2
NUM_SUBCORES = 16
NW = NUM_CORES * NUM_SUBCORES
B_PER_W = B_TOTAL // NW
CHUNK = 64
N_CHUNKS = B_PER_W // CHUNK
NBUF = 3


def _gather_body(ta_hbm, tb_hbm, idx_hbm, out_hbm, idx_v, rows_n, rows_t, *sems):
    gasems = sems[0:NBUF]
    gbsems = sems[NBUF:2 * NBUF]
    wsems = sems[2 * NBUF:3 * NBUF]
    wid = lax.axis_index("s") * NUM_CORES + lax.axis_index("c")
    base = wid * B_PER_W

    pltpu.sync_copy(idx_hbm.at[pl.ds(base, B_PER_W)], idx_v)

    def start_gathers(g, b):
        idx_slice = idx_v.at[pl.ds(g * CHUNK, CHUNK)]
        pltpu.async_copy(
            ta_hbm.at[idx_slice], rows_n.at[b, :, pl.ds(0, SPLIT)], gasems[b]
        )
        pltpu.async_copy(tb_hbm.at[idx_slice], rows_t.at[b], gbsems[b])

    def wait_gathers(g, b):
        idx_slice = idx_v.at[pl.ds(g * CHUNK, CHUNK)]
        pltpu.make_async_copy(
            ta_hbm.at[idx_slice], rows_n.at[b, :, pl.ds(0, SPLIT)], gasems[b]
        ).wait()
        pltpu.make_async_copy(
            tb_hbm.at[idx_slice], rows_t.at[b], gbsems[b]
        ).wait()

    def wait_wb(b):
        pltpu.make_async_copy(
            rows_n.at[b], out_hbm.at[pl.ds(base, CHUNK)], wsems[b]
        ).wait()

    for j in range(NBUF - 1):
        start_gathers(j, j)

    def outer(i, carry):
        for b in range(NBUF):
            g = i * NBUF + b
            pb = (b + NBUF - 1) % NBUF
            pf = g + NBUF - 1

            @pl.when(pf < N_CHUNKS)
            def _():
                @pl.when(g >= 1)
                def _():
                    wait_wb(pb)

                start_gathers(pf, pb)

            wait_gathers(g, b)

            def row_body(r, carry2):
                rows_n[b, r, pl.ds(SPLIT, 16)] = rows_t[b, r, pl.ds(0, 16)]
                rows_n[b, r, pl.ds(SPLIT + 16, 16)] = rows_t[b, r, pl.ds(16, 16)]
                lanes = lax.iota(jnp.int32, 16)
                vals = rows_t[b, r, pl.ds(32, 16)]
                plsc.store_scatter(
                    rows_n,
                    [jnp.full((16,), b, jnp.int32),
                     jnp.full((16,), r, jnp.int32),
                     SPLIT + 32 + lanes],
                    vals,
                    mask=lanes < TAIL - 32,
                )
                return carry2

            lax.fori_loop(0, CHUNK, row_body, 0)

            pltpu.async_copy(
                rows_n.at[b], out_hbm.at[pl.ds(base + g * CHUNK, CHUNK)], wsems[b]
            )
        return carry

    lax.fori_loop(0, N_CHUNKS // NBUF, outer, 0)

    for g in range((N_CHUNKS // NBUF) * NBUF, N_CHUNKS):
        b = g % NBUF
        wait_gathers(g, b)

        def row_body(r, carry2):
            rows_n[b, r, pl.ds(SPLIT, 16)] = rows_t[b, r, pl.ds(0, 16)]
            rows_n[b, r, pl.ds(SPLIT + 16, 16)] = rows_t[b, r, pl.ds(16, 16)]
            lanes = lax.iota(jnp.int32, 16)
            vals = rows_t[b, r, pl.ds(32, 16)]
            plsc.store_scatter(
                rows_n,
                [jnp.full((16,), b, jnp.int32),
                 jnp.full((16,), r, jnp.int32),
                 SPLIT + 32 + lanes],
                vals,
                mask=lanes < TAIL - 32,
            )
            return carry2

        lax.fori_loop(0, CHUNK, row_body, 0)
        pltpu.async_copy(
            rows_n.at[b], out_hbm.at[pl.ds(base + g * CHUNK, CHUNK)], wsems[b]
        )

    for b in range(NBUF):
        pltpu.make_async_copy(
            rows_n.at[b], out_hbm.at[pl.ds(base, CHUNK)], wsems[b]
        ).wait()


@jax.jit
def _sc_gather(table_a, table_b, idx_flat):
    k = functools.partial(
        pl.kernel,
        out_type=jax.ShapeDtypeStruct((B_TOTAL, EMBED), jnp.float32),
        mesh=plsc.VectorSubcoreMesh(core_axis_name="c", subcore_axis_name="s"),
        scratch_types=[
            pltpu.VMEM((B_PER_W,), jnp.int32),
            pltpu.VMEM((NBUF, CHUNK, EMBED), jnp.float32),
            pltpu.VMEM((NBUF, CHUNK, TAIL_PAD), jnp.float32),
        ] + [pltpu.SemaphoreType.DMA] * (3 * NBUF),
        compiler_params=pltpu.CompilerParams(needs_layout_passes=False),
    )(_gather_body)
    return k(table_a, table_b, idx_flat)


def kernel(table, x):
    idx_flat = x.reshape(B_TOTAL)
    table_a = table[:, :SPLIT]
    table_b = jnp.pad(table[:, SPLIT:], ((0, 0), (0, TAIL_PAD - TAIL)))
    out = _sc_gather(table_a, table_b, idx_flat)
    return out.reshape(BATCH, HIST, EMBED)


# speedup vs baseline: 1.1382x; 1.0013x over previous
import functools

import jax
import jax.numpy as jnp
from jax import lax
from jax.experimental import pallas as pl
from jax.experimental.pallas import tpu as pltpu
from jax.experimental.pallas import tpu_sc as plsc

VOCAB = 1000
EMBED = 300
BATCH = 1024
HIST = 200

SPLIT = 256
TAIL = EMBED - SPLIT
TAIL_PAD = 128

B_TOTAL = BATCH * HIST
NUM_CORES = 2
NUM_SUBCORES = 16
NW = NUM_CORES * NUM_SUBCORES
B_PER_W = B_TOTAL // NW
CHUNK = 64
N_CHUNKS = B_PER_W // CHUNK
NBUF = 3


def _gather_body(ta_hbm, tb_hbm, idx_hbm, out_hbm, idx_v, rows_n, rows_t, *sems):
    gasems = sems[0:NBUF]
    gbsems = sems[NBUF:2 * NBUF]
    wsems = sems[2 * NBUF:3 * NBUF]
    wid = lax.axis_index("s") * NUM_CORES + lax.axis_index("c")
    base = wid * B_PER_W

    pltpu.sync_copy(idx_hbm.at[pl.ds(base, B_PER_W)], idx_v)

    def start_gathers(g, b):
        idx_slice = idx_v.at[pl.ds(g * CHUNK, CHUNK)]
        pltpu.async_copy(
            ta_hbm.at[idx_slice], rows_n.at[b, :, pl.ds(0, SPLIT)], gasems[b]
        )
        pltpu.async_copy(tb_hbm.at[idx_slice], rows_t.at[b], gbsems[b])

    def wait_gather_a(g, b):
        idx_slice = idx_v.at[pl.ds(g * CHUNK, CHUNK)]
        pltpu.make_async_copy(
            ta_hbm.at[idx_slice], rows_n.at[b, :, pl.ds(0, SPLIT)], gasems[b]
        ).wait()

    def wait_gather_b(g, b):
        idx_slice = idx_v.at[pl.ds(g * CHUNK, CHUNK)]
        pltpu.make_async_copy(
            tb_hbm.at[idx_slice], rows_t.at[b], gbsems[b]
        ).wait()

    def wait_wb(b):
        pltpu.make_async_copy(
            rows_n.at[b], out_hbm.at[pl.ds(base, CHUNK)], wsems[b]
        ).wait()

    for j in range(NBUF - 1):
        start_gathers(j, j)

    def outer(i, carry):
        for b in range(NBUF):
            g = i * NBUF + b
            pb = (b + NBUF - 1) % NBUF
            pf = g + NBUF - 1

            @pl.when(pf < N_CHUNKS)
            def _():
                @pl.when(g >= 1)
                def _():
                    wait_wb(pb)

                start_gathers(pf, pb)

            wait_gather_b(g, b)

            def row_body(r, carry2):
                rows_n[b, r, pl.ds(SPLIT, 16)] = rows_t[b, r, pl.ds(0, 16)]
                rows_n[b, r, pl.ds(SPLIT + 16, 16)] = rows_t[b, r, pl.ds(16, 16)]
                lanes = lax.iota(jnp.int32, 16)
                vals = rows_t[b, r, pl.ds(32, 16)]
                plsc.store_scatter(
                    rows_n,
                    [jnp.full((16,), b, jnp.int32),
                     jnp.full((16,), r, jnp.int32),
                     SPLIT + 32 + lanes],
                    vals,
                    mask=lanes < TAIL - 32,
                )
                return carry2

            lax.fori_loop(0, CHUNK, row_body, 0)

            wait_gather_a(g, b)

            pltpu.async_copy(
                rows_n.at[b], out_hbm.at[pl.ds(base + g * CHUNK, CHUNK)], wsems[b]
            )
        return carry

    lax.fori_loop(0, N_CHUNKS // NBUF, outer, 0)

    for g in range((N_CHUNKS // NBUF) * NBUF, N_CHUNKS):
        b = g % NBUF
        wait_gather_b(g, b)

        def row_body(r, carry2):
            rows_n[b, r, pl.ds(SPLIT, 16)] = rows_t[b, r, pl.ds(0, 16)]
            rows_n[b, r, pl.ds(SPLIT + 16, 16)] = rows_t[b, r, pl.ds(16, 16)]
            lanes = lax.iota(jnp.int32, 16)
            vals = rows_t[b, r, pl.ds(32, 16)]
            plsc.store_scatter(
                rows_n,
                [jnp.full((16,), b, jnp.int32),
                 jnp.full((16,), r, jnp.int32),
                 SPLIT + 32 + lanes],
                vals,
                mask=lanes < TAIL - 32,
            )
            return carry2

        lax.fori_loop(0, CHUNK, row_body, 0)
        wait_gather_a(g, b)
        pltpu.async_copy(
            rows_n.at[b], out_hbm.at[pl.ds(base + g * CHUNK, CHUNK)], wsems[b]
        )

    for b in range(NBUF):
        pltpu.make_async_copy(
            rows_n.at[b], out_hbm.at[pl.ds(base, CHUNK)], wsems[b]
        ).wait()


@jax.jit
def _sc_gather(table_a, table_b, idx_flat):
    k = functools.partial(
        pl.kernel,
        out_type=jax.ShapeDtypeStruct((B_TOTAL, EMBED), jnp.float32),
        mesh=plsc.VectorSubcoreMesh(core_axis_name="c", subcore_axis_name="s"),
        scratch_types=[
            pltpu.VMEM((B_PER_W,), jnp.int32),
            pltpu.VMEM((NBUF, CHUNK, EMBED), jnp.float32),
            pltpu.VMEM((NBUF, CHUNK, TAIL_PAD), jnp.float32),
        ] + [pltpu.SemaphoreType.DMA] * (3 * NBUF),
        compiler_params=pltpu.CompilerParams(needs_layout_passes=False),
    )(_gather_body)
    return k(table_a, table_b, idx_flat)


def kernel(table, x):
    idx_flat = x.reshape(B_TOTAL)
    table_a = table[:, :SPLIT]
    table_b = jnp.pad(table[:, SPLIT:], ((0, 0), (0, TAIL_PAD - TAIL)))
    out = _sc_gather(table_a, table_b, idx_flat)
    return out.reshape(BATCH, HIST, EMBED)
